# baseline (speedup 1.0000x reference)
import functools

import jax
import jax.numpy as jnp
from jax.experimental import pallas as pl
from jax.experimental.pallas import tpu as pltpu

HID = 128
TOPK = 8192
GK = 32
SIG = 0.1
MINS = 1e-06


def _node_mlp_body(x_ref, w1_ref, b1_ref, w2_ref, b2_ref, o_ref):
    x = x_ref[...]
    t = jax.nn.relu(jnp.dot(x, w1_ref[...], preferred_element_type=jnp.float32) + b1_ref[...])
    o_ref[...] = jax.nn.relu(jnp.dot(t, w2_ref[...], preferred_element_type=jnp.float32) + b2_ref[...])


def _node_mlp(x, w1, b1, w2, b2):
    R = x.shape[0]
    BR = 1024
    return pl.pallas_call(
        _node_mlp_body,
        grid=(R // BR,),
        in_specs=[
            pl.BlockSpec((BR, x.shape[1]), lambda i: (i, 0)),
            pl.BlockSpec(w1.shape, lambda i: (0, 0)),
            pl.BlockSpec(b1.shape, lambda i: (0,)),
            pl.BlockSpec(w2.shape, lambda i: (0, 0)),
            pl.BlockSpec(b2.shape, lambda i: (0,)),
        ],
        out_specs=pl.BlockSpec((BR, w2.shape[1]), lambda i: (i, 0)),
        out_shape=jax.ShapeDtypeStruct((R, w2.shape[1]), jnp.float32),
    )(x, w1, b1, w2, b2)


def _edge_agg_body(hn_ref, compat_ref, resid_ref, w1_ref,
                   b1_ref, w2_ref, b2_ref, agg_ref):
    w1 = w1_ref[...]
    w2 = w2_ref[...]
    b1 = b1_ref[...]
    b2 = b2_ref[...]
    acc = jnp.zeros(agg_ref.shape, jnp.float32)
    for j in range(GK):
        cj = compat_ref[j]
        edge_in = jnp.concatenate([hn_ref[j], cj, resid_ref[j]], axis=1)
        t = jax.nn.relu(jnp.dot(edge_in, w1, preferred_element_type=jnp.float32) + b1)
        msg = jax.nn.relu(jnp.dot(t, w2, preferred_element_type=jnp.float32) + b2)
        acc = acc + msg * cj
    agg_ref[...] = acc * (1.0 / GK)


def _edge_agg(h_nbr_t, compat_t, resid_t, ew1, eb1, ew2, eb2):
    N = h_nbr_t.shape[1]
    BR = 256
    return pl.pallas_call(
        _edge_agg_body,
        grid=(N // BR,),
        in_specs=[
            pl.BlockSpec((GK, BR, HID), lambda i: (0, i, 0)),
            pl.BlockSpec((GK, BR, 1), lambda i: (0, i, 0)),
            pl.BlockSpec((GK, BR, 1), lambda i: (0, i, 0)),
            pl.BlockSpec(ew1.shape, lambda i: (0, 0)),
            pl.BlockSpec(eb1.shape, lambda i: (0,)),
            pl.BlockSpec(ew2.shape, lambda i: (0, 0)),
            pl.BlockSpec(eb2.shape, lambda i: (0,)),
        ],
        out_specs=pl.BlockSpec((BR, HID), lambda i: (i, 0)),
        out_shape=jax.ShapeDtypeStruct((N, HID), jnp.float32),
    )(h_nbr_t, compat_t, resid_t, ew1, eb1, ew2, eb2)


def _update_gate_body(h_ref, agg_ref, uw1_ref, ub1_ref, uw2_ref, ub2_ref,
                      ow1_ref, ob1_ref, ow2_ref, ob2_ref, hout_ref, gate_ref):
    h = h_ref[...]
    agg = agg_ref[...]
    ha = jnp.concatenate([h, agg], axis=1)
    t = jax.nn.relu(jnp.dot(ha, uw1_ref[...], preferred_element_type=jnp.float32)
                    + ub1_ref[...])
    hn = h + jnp.dot(t, uw2_ref[...], preferred_element_type=jnp.float32) + ub2_ref[...]
    hout_ref[...] = hn
    g = jax.nn.relu(jnp.dot(hn, ow1_ref[...], preferred_element_type=jnp.float32) + ob1_ref[...])
    gate_ref[...] = jax.nn.sigmoid(jnp.dot(g, ow2_ref[...], preferred_element_type=jnp.float32) + ob2_ref[...])


def _update_gate(h, agg, uw1, ub1, uw2, ub2, ow1, ob1, ow2, ob2):
    N = h.shape[0]
    BR = 1024
    return pl.pallas_call(
        _update_gate_body,
        grid=(N // BR,),
        in_specs=[
            pl.BlockSpec((BR, HID), lambda i: (i, 0)),
            pl.BlockSpec((BR, HID), lambda i: (i, 0)),
            pl.BlockSpec(uw1.shape, lambda i: (0, 0)),
            pl.BlockSpec(ub1.shape, lambda i: (0,)),
            pl.BlockSpec(uw2.shape, lambda i: (0, 0)),
            pl.BlockSpec(ub2.shape, lambda i: (0,)),
            pl.BlockSpec(ow1.shape, lambda i: (0, 0)),
            pl.BlockSpec(ob1.shape, lambda i: (0,)),
            pl.BlockSpec(ow2.shape, lambda i: (0, 0)),
            pl.BlockSpec(ob2.shape, lambda i: (0,)),
        ],
        out_specs=[
            pl.BlockSpec((BR, HID), lambda i: (i, 0)),
            pl.BlockSpec((BR, 1), lambda i: (i, 0)),
        ],
        out_shape=[
            jax.ShapeDtypeStruct((N, HID), jnp.float32),
            jax.ShapeDtypeStruct((N, 1), jnp.float32),
        ],
    )(h, agg, uw1, ub1, uw2, ub2, ow1, ob1, ow2, ob2)


def _knn_body(rpb_ref, rpat_ref, out_ref, d_ref):
    rpb = rpb_ref[...]
    rpat = rpat_ref[...]
    sqb = jnp.sum(rpb * rpb, axis=1, keepdims=True)
    sqa = jnp.sum(rpat * rpat, axis=0, keepdims=True)
    dots = jnp.dot(rpb, rpat, preferred_element_type=jnp.float32)
    d2 = jnp.clip(sqb + sqa - 2.0 * dots, 0.0, None)
    d_ref[...] = jnp.sqrt(d2)
    br, n = d_ref.shape
    iota = jax.lax.broadcasted_iota(jnp.int32, (br, n), 1)
    for k in range(GK + 1):
        d = d_ref[...]
        m = jnp.min(d, axis=1, keepdims=True)
        idx = jnp.min(jnp.where(d == m, iota, n), axis=1, keepdims=True)
        if k > 0:
            out_ref[:, k - 1:k] = idx
        d_ref[...] = jnp.where(iota == idx, jnp.inf, d)


def _knn(ref_pts):
    N = ref_pts.shape[0]
    BR = 256
    rpat = ref_pts.T
    return pl.pallas_call(
        _knn_body,
        grid=(N // BR,),
        in_specs=[
            pl.BlockSpec((BR, 3), lambda i: (i, 0)),
            pl.BlockSpec((3, N), lambda i: (0, 0)),
        ],
        out_specs=pl.BlockSpec((BR, GK), lambda i: (i, 0)),
        out_shape=jax.ShapeDtypeStruct((N, GK), jnp.int32),
        scratch_shapes=[pltpu.VMEM((BR, N), jnp.float32)],
    )(ref_pts, rpat)


def kernel(ref_node_corr_indices, src_node_corr_indices, node_corr_scores,
           ref_points_c, src_points_c, ref_feats_c, src_feats_c,
           nw1, nb1, nw2, nb2, ew1, eb1, ew2, eb2,
           uw1, ub1, uw2, ub2, ow1, ob1, ow2, ob2):
    keep = TOPK
    top_scores, top_ids = jax.lax.top_k(node_corr_scores, keep)
    ref_idx = ref_node_corr_indices[top_ids]
    src_idx = src_node_corr_indices[top_ids]
    ref_pts = ref_points_c[ref_idx]
    src_pts = src_points_c[src_idx]
    ref_f = ref_feats_c[ref_idx]
    src_f = src_feats_c[src_idx]

    num = jnp.sum(ref_f * src_f, axis=-1)
    den = jnp.maximum(jnp.linalg.norm(ref_f, axis=-1), 1e-08) * jnp.maximum(jnp.linalg.norm(src_f, axis=-1), 1e-08)
    feat_cos = (num / den)[:, None]
    feat_l2 = jnp.linalg.norm(ref_f - src_f, axis=-1, keepdims=True)
    score = jnp.clip(top_scores, MINS, None)[:, None]
    log_score = jnp.log(jnp.clip(score, MINS, None))
    node_x = jnp.concatenate([score, log_score, feat_cos, feat_l2], axis=1)
    h = _node_mlp(node_x, nw1, nb1, nw2, nb2)

    knn_ids = _knn(ref_pts)

    ref_nbr = ref_pts[knn_ids]
    src_nbr = src_pts[knn_ids]
    rel = jnp.linalg.norm(ref_pts[:, None, :] - ref_nbr, axis=-1)
    sel = jnp.linalg.norm(src_pts[:, None, :] - src_nbr, axis=-1)
    residual = jnp.abs(rel - sel)
    compat = jnp.exp(-residual ** 2 / (2.0 * SIG ** 2 + 1e-08))
    h_nbr_t = h[knn_ids.T]
    agg = _edge_agg(h_nbr_t, compat.T[:, :, None], residual.T[:, :, None],
                    ew1, eb1, ew2, eb2)
    h, gate2 = _update_gate(h, agg, uw1, ub1, uw2, ub2, ow1, ob1, ow2, ob2)
    gate = gate2[:, 0]

    mean_compat = compat.mean(axis=1)

    refined = jnp.clip(top_scores, MINS, None) * (0.5 * gate + 0.5 * mean_compat)
    refined = jnp.clip(refined, MINS, None)
    order = jnp.argsort(-refined)
    return (ref_idx[order], src_idx[order], refined[order])


# speedup vs baseline: 1.1635x; 1.1635x over previous
import functools

import jax
import jax.numpy as jnp
from jax.experimental import pallas as pl
from jax.experimental.pallas import tpu as pltpu

HID = 128
TOPK = 8192
GK = 32
SIG = 0.1
MINS = 1e-06


def _node_mlp_body(x_ref, w1_ref, b1_ref, w2_ref, b2_ref, o_ref):
    x = x_ref[...]
    t = jax.nn.relu(jnp.dot(x, w1_ref[...], preferred_element_type=jnp.float32) + b1_ref[...])
    o_ref[...] = jax.nn.relu(jnp.dot(t, w2_ref[...], preferred_element_type=jnp.float32) + b2_ref[...])


def _node_mlp(x, w1, b1, w2, b2):
    R = x.shape[0]
    BR = 1024
    return pl.pallas_call(
        _node_mlp_body,
        grid=(R // BR,),
        in_specs=[
            pl.BlockSpec((BR, x.shape[1]), lambda i: (i, 0)),
            pl.BlockSpec(w1.shape, lambda i: (0, 0)),
            pl.BlockSpec(b1.shape, lambda i: (0,)),
            pl.BlockSpec(w2.shape, lambda i: (0, 0)),
            pl.BlockSpec(b2.shape, lambda i: (0,)),
        ],
        out_specs=pl.BlockSpec((BR, w2.shape[1]), lambda i: (i, 0)),
        out_shape=jax.ShapeDtypeStruct((R, w2.shape[1]), jnp.float32),
    )(x, w1, b1, w2, b2)


def _edge_agg_body(hn_ref, compat_ref, resid_ref, w1_ref,
                   b1_ref, w2_ref, b2_ref, agg_ref):
    w1 = w1_ref[...]
    w2 = w2_ref[...]
    b1 = b1_ref[...]
    b2 = b2_ref[...]
    acc = jnp.zeros(agg_ref.shape, jnp.float32)
    for j in range(GK):
        cj = compat_ref[j]
        edge_in = jnp.concatenate([hn_ref[j], cj, resid_ref[j]], axis=1)
        t = jax.nn.relu(jnp.dot(edge_in, w1, preferred_element_type=jnp.float32) + b1)
        msg = jax.nn.relu(jnp.dot(t, w2, preferred_element_type=jnp.float32) + b2)
        acc = acc + msg * cj
    agg_ref[...] = acc * (1.0 / GK)


def _edge_agg(h_nbr_t, compat_t, resid_t, ew1, eb1, ew2, eb2):
    N = h_nbr_t.shape[1]
    BR = 256
    return pl.pallas_call(
        _edge_agg_body,
        grid=(N // BR,),
        in_specs=[
            pl.BlockSpec((GK, BR, HID), lambda i: (0, i, 0)),
            pl.BlockSpec((GK, BR, 1), lambda i: (0, i, 0)),
            pl.BlockSpec((GK, BR, 1), lambda i: (0, i, 0)),
            pl.BlockSpec(ew1.shape, lambda i: (0, 0)),
            pl.BlockSpec(eb1.shape, lambda i: (0,)),
            pl.BlockSpec(ew2.shape, lambda i: (0, 0)),
            pl.BlockSpec(eb2.shape, lambda i: (0,)),
        ],
        out_specs=pl.BlockSpec((BR, HID), lambda i: (i, 0)),
        out_shape=jax.ShapeDtypeStruct((N, HID), jnp.float32),
    )(h_nbr_t, compat_t, resid_t, ew1, eb1, ew2, eb2)


def _update_gate_body(h_ref, agg_ref, uw1_ref, ub1_ref, uw2_ref, ub2_ref,
                      ow1_ref, ob1_ref, ow2_ref, ob2_ref, hout_ref, gate_ref):
    h = h_ref[...]
    agg = agg_ref[...]
    ha = jnp.concatenate([h, agg], axis=1)
    t = jax.nn.relu(jnp.dot(ha, uw1_ref[...], preferred_element_type=jnp.float32)
                    + ub1_ref[...])
    hn = h + jnp.dot(t, uw2_ref[...], preferred_element_type=jnp.float32) + ub2_ref[...]
    hout_ref[...] = hn
    g = jax.nn.relu(jnp.dot(hn, ow1_ref[...], preferred_element_type=jnp.float32) + ob1_ref[...])
    gate_ref[...] = jax.nn.sigmoid(jnp.dot(g, ow2_ref[...], preferred_element_type=jnp.float32) + ob2_ref[...])


def _update_gate(h, agg, uw1, ub1, uw2, ub2, ow1, ob1, ow2, ob2):
    N = h.shape[0]
    BR = 1024
    return pl.pallas_call(
        _update_gate_body,
        grid=(N // BR,),
        in_specs=[
            pl.BlockSpec((BR, HID), lambda i: (i, 0)),
            pl.BlockSpec((BR, HID), lambda i: (i, 0)),
            pl.BlockSpec(uw1.shape, lambda i: (0, 0)),
            pl.BlockSpec(ub1.shape, lambda i: (0,)),
            pl.BlockSpec(uw2.shape, lambda i: (0, 0)),
            pl.BlockSpec(ub2.shape, lambda i: (0,)),
            pl.BlockSpec(ow1.shape, lambda i: (0, 0)),
            pl.BlockSpec(ob1.shape, lambda i: (0,)),
            pl.BlockSpec(ow2.shape, lambda i: (0, 0)),
            pl.BlockSpec(ob2.shape, lambda i: (0,)),
        ],
        out_specs=[
            pl.BlockSpec((BR, HID), lambda i: (i, 0)),
            pl.BlockSpec((BR, 1), lambda i: (i, 0)),
        ],
        out_shape=[
            jax.ShapeDtypeStruct((N, HID), jnp.float32),
            jax.ShapeDtypeStruct((N, 1), jnp.float32),
        ],
    )(h, agg, uw1, ub1, uw2, ub2, ow1, ob1, ow2, ob2)


def _knn_body(rpb_ref, rpat_ref, out_ref, d_ref):
    rpb = rpb_ref[...]
    rpat = rpat_ref[...]
    sqb = jnp.sum(rpb * rpb, axis=1, keepdims=True)
    sqa = jnp.sum(rpat * rpat, axis=0, keepdims=True)
    dots = jnp.dot(rpb, rpat, preferred_element_type=jnp.float32)
    d2 = jnp.clip(sqb + sqa - 2.0 * dots, 0.0, None)
    d_ref[...] = jnp.sqrt(d2)
    br, n = d_ref.shape
    iota = jax.lax.broadcasted_iota(jnp.int32, (br, n), 1)
    for k in range(GK + 1):
        d = d_ref[...]
        m = jnp.min(d, axis=1, keepdims=True)
        idx = jnp.min(jnp.where(d == m, iota, n), axis=1, keepdims=True)
        if k > 0:
            out_ref[:, k - 1:k] = idx
        d_ref[...] = jnp.where(iota == idx, jnp.inf, d)


def _knn(ref_pts):
    N = ref_pts.shape[0]
    BR = 256
    rpat = ref_pts.T
    return pl.pallas_call(
        _knn_body,
        grid=(N // BR,),
        in_specs=[
            pl.BlockSpec((BR, 3), lambda i: (i, 0)),
            pl.BlockSpec((3, N), lambda i: (0, 0)),
        ],
        out_specs=pl.BlockSpec((BR, GK), lambda i: (i, 0)),
        out_shape=jax.ShapeDtypeStruct((N, GK), jnp.int32),
        scratch_shapes=[pltpu.VMEM((BR, N), jnp.float32)],
    )(ref_pts, rpat)


def kernel(ref_node_corr_indices, src_node_corr_indices, node_corr_scores,
           ref_points_c, src_points_c, ref_feats_c, src_feats_c,
           nw1, nb1, nw2, nb2, ew1, eb1, ew2, eb2,
           uw1, ub1, uw2, ub2, ow1, ob1, ow2, ob2):
    keep = TOPK
    top_scores, top_ids = jax.lax.top_k(node_corr_scores, keep)
    ref_idx = ref_node_corr_indices[top_ids]
    src_idx = src_node_corr_indices[top_ids]
    ref_pts = ref_points_c[ref_idx]
    src_pts = src_points_c[src_idx]
    ref_f = ref_feats_c[ref_idx]
    src_f = src_feats_c[src_idx]

    num = jnp.sum(ref_f * src_f, axis=-1)
    den = jnp.maximum(jnp.linalg.norm(ref_f, axis=-1), 1e-08) * jnp.maximum(jnp.linalg.norm(src_f, axis=-1), 1e-08)
    feat_cos = (num / den)[:, None]
    feat_l2 = jnp.linalg.norm(ref_f - src_f, axis=-1, keepdims=True)
    score = jnp.clip(top_scores, MINS, None)[:, None]
    log_score = jnp.log(jnp.clip(score, MINS, None))
    node_x = jnp.concatenate([score, log_score, feat_cos, feat_l2], axis=1)
    h = _node_mlp(node_x, nw1, nb1, nw2, nb2)

    knn_ids = _knn(ref_pts)

    pts_cat = jnp.concatenate([ref_pts, src_pts, jnp.zeros((TOPK, 10), jnp.float32)], axis=1)
    nbr = pts_cat[knn_ids]
    rel = jnp.linalg.norm(ref_pts[:, None, :] - nbr[..., 0:3], axis=-1)
    sel = jnp.linalg.norm(src_pts[:, None, :] - nbr[..., 3:6], axis=-1)
    residual = jnp.abs(rel - sel)
    compat = jnp.exp(-residual ** 2 / (2.0 * SIG ** 2 + 1e-08))
    h_nbr_t = h[knn_ids.T]
    agg = _edge_agg(h_nbr_t, compat.T[:, :, None], residual.T[:, :, None],
                    ew1, eb1, ew2, eb2)
    h, gate2 = _update_gate(h, agg, uw1, ub1, uw2, ub2, ow1, ob1, ow2, ob2)
    gate = gate2[:, 0]

    mean_compat = compat.mean(axis=1)

    refined = jnp.clip(top_scores, MINS, None) * (0.5 * gate + 0.5 * mean_compat)
    refined = jnp.clip(refined, MINS, None)
    order = jnp.argsort(-refined)
    return (ref_idx[order], src_idx[order], refined[order])


# speedup vs baseline: 1.4073x; 1.2095x over previous
import functools

import jax
import jax.numpy as jnp
from jax.experimental import pallas as pl
from jax.experimental.pallas import tpu as pltpu
from jax.experimental.pallas import tpu_sc as plsc

HID = 128
TOPK = 8192
GK = 32
SIG = 0.1
MINS = 1e-06


def _node_mlp_body(x_ref, w1_ref, b1_ref, w2_ref, b2_ref, o_ref):
    x = x_ref[...]
    t = jax.nn.relu(jnp.dot(x, w1_ref[...], preferred_element_type=jnp.float32) + b1_ref[...])
    o_ref[...] = jax.nn.relu(jnp.dot(t, w2_ref[...], preferred_element_type=jnp.float32) + b2_ref[...])


def _node_mlp(x, w1, b1, w2, b2):
    R = x.shape[0]
    BR = 1024
    return pl.pallas_call(
        _node_mlp_body,
        grid=(R // BR,),
        in_specs=[
            pl.BlockSpec((BR, x.shape[1]), lambda i: (i, 0)),
            pl.BlockSpec(w1.shape, lambda i: (0, 0)),
            pl.BlockSpec(b1.shape, lambda i: (0,)),
            pl.BlockSpec(w2.shape, lambda i: (0, 0)),
            pl.BlockSpec(b2.shape, lambda i: (0,)),
        ],
        out_specs=pl.BlockSpec((BR, w2.shape[1]), lambda i: (i, 0)),
        out_shape=jax.ShapeDtypeStruct((R, w2.shape[1]), jnp.float32),
    )(x, w1, b1, w2, b2)


def _edge_agg_body(hn_ref, compat_ref, resid_ref, w1_ref,
                   b1_ref, w2_ref, b2_ref, agg_ref):
    w1 = w1_ref[...]
    w2 = w2_ref[...]
    b1 = b1_ref[...]
    b2 = b2_ref[...]
    acc = jnp.zeros(agg_ref.shape, jnp.float32)
    for j in range(GK):
        cj = compat_ref[j]
        edge_in = jnp.concatenate([hn_ref[j], cj, resid_ref[j]], axis=1)
        t = jax.nn.relu(jnp.dot(edge_in, w1, preferred_element_type=jnp.float32) + b1)
        msg = jax.nn.relu(jnp.dot(t, w2, preferred_element_type=jnp.float32) + b2)
        acc = acc + msg * cj
    agg_ref[...] = acc * (1.0 / GK)


def _edge_agg(h_nbr_t, compat_t, resid_t, ew1, eb1, ew2, eb2):
    N = h_nbr_t.shape[1]
    BR = 256
    return pl.pallas_call(
        _edge_agg_body,
        grid=(N // BR,),
        in_specs=[
            pl.BlockSpec((GK, BR, HID), lambda i: (0, i, 0)),
            pl.BlockSpec((GK, BR, 1), lambda i: (0, i, 0)),
            pl.BlockSpec((GK, BR, 1), lambda i: (0, i, 0)),
            pl.BlockSpec(ew1.shape, lambda i: (0, 0)),
            pl.BlockSpec(eb1.shape, lambda i: (0,)),
            pl.BlockSpec(ew2.shape, lambda i: (0, 0)),
            pl.BlockSpec(eb2.shape, lambda i: (0,)),
        ],
        out_specs=pl.BlockSpec((BR, HID), lambda i: (i, 0)),
        out_shape=jax.ShapeDtypeStruct((N, HID), jnp.float32),
    )(h_nbr_t, compat_t, resid_t, ew1, eb1, ew2, eb2)


def _update_gate_body(h_ref, agg_ref, uw1_ref, ub1_ref, uw2_ref, ub2_ref,
                      ow1_ref, ob1_ref, ow2_ref, ob2_ref, hout_ref, gate_ref):
    h = h_ref[...]
    agg = agg_ref[...]
    ha = jnp.concatenate([h, agg], axis=1)
    t = jax.nn.relu(jnp.dot(ha, uw1_ref[...], preferred_element_type=jnp.float32)
                    + ub1_ref[...])
    hn = h + jnp.dot(t, uw2_ref[...], preferred_element_type=jnp.float32) + ub2_ref[...]
    hout_ref[...] = hn
    g = jax.nn.relu(jnp.dot(hn, ow1_ref[...], preferred_element_type=jnp.float32) + ob1_ref[...])
    gate_ref[...] = jax.nn.sigmoid(jnp.dot(g, ow2_ref[...], preferred_element_type=jnp.float32) + ob2_ref[...])


def _update_gate(h, agg, uw1, ub1, uw2, ub2, ow1, ob1, ow2, ob2):
    N = h.shape[0]
    BR = 1024
    return pl.pallas_call(
        _update_gate_body,
        grid=(N // BR,),
        in_specs=[
            pl.BlockSpec((BR, HID), lambda i: (i, 0)),
            pl.BlockSpec((BR, HID), lambda i: (i, 0)),
            pl.BlockSpec(uw1.shape, lambda i: (0, 0)),
            pl.BlockSpec(ub1.shape, lambda i: (0,)),
            pl.BlockSpec(uw2.shape, lambda i: (0, 0)),
            pl.BlockSpec(ub2.shape, lambda i: (0,)),
            pl.BlockSpec(ow1.shape, lambda i: (0, 0)),
            pl.BlockSpec(ob1.shape, lambda i: (0,)),
            pl.BlockSpec(ow2.shape, lambda i: (0, 0)),
            pl.BlockSpec(ob2.shape, lambda i: (0,)),
        ],
        out_specs=[
            pl.BlockSpec((BR, HID), lambda i: (i, 0)),
            pl.BlockSpec((BR, 1), lambda i: (i, 0)),
        ],
        out_shape=[
            jax.ShapeDtypeStruct((N, HID), jnp.float32),
            jax.ShapeDtypeStruct((N, 1), jnp.float32),
        ],
    )(h, agg, uw1, ub1, uw2, ub2, ow1, ob1, ow2, ob2)


def _sc_gather_rows(table, idx):
    V, D = table.shape
    B = idx.shape[0]
    info = plsc.get_sparse_core_info()
    NC, NS = info.num_cores, info.num_subcores
    NW = NC * NS
    b_per_w = B // NW
    CH = 128
    n_ch = b_per_w // CH
    mesh = plsc.VectorSubcoreMesh(core_axis_name="c", subcore_axis_name="s")

    @functools.partial(
        pl.kernel, mesh=mesh,
        out_type=jax.ShapeDtypeStruct((B, D), jnp.float32),
        scratch_types=[
            pltpu.VMEM((CH,), jnp.int32),
            pltpu.VMEM((CH, D), jnp.float32),
            pltpu.SemaphoreType.DMA,
        ],
    )
    def k(table_hbm, idx_hbm, out_hbm, idx_v, rows_v, sem):
        wid = jax.lax.axis_index("s") * NC + jax.lax.axis_index("c")
        base = wid * b_per_w

        def body(c, carry):
            off = base + c * CH
            pltpu.sync_copy(idx_hbm.at[pl.ds(off, CH)], idx_v)
            pltpu.async_copy(table_hbm.at[idx_v], rows_v, sem).wait()
            pltpu.sync_copy(rows_v, out_hbm.at[pl.ds(off, CH)])
            return carry

        jax.lax.fori_loop(0, n_ch, body, 0)

    return k(table, idx)


def _knn_body(rpb_ref, rpat_ref, out_ref, d_ref):
    rpb = rpb_ref[...]
    rpat = rpat_ref[...]
    sqb = jnp.sum(rpb * rpb, axis=1, keepdims=True)
    sqa = jnp.sum(rpat * rpat, axis=0, keepdims=True)
    dots = jnp.dot(rpb, rpat, preferred_element_type=jnp.float32)
    d2 = jnp.clip(sqb + sqa - 2.0 * dots, 0.0, None)
    d_ref[...] = jnp.sqrt(d2)
    br, n = d_ref.shape
    iota = jax.lax.broadcasted_iota(jnp.int32, (br, n), 1)
    for k in range(GK + 1):
        d = d_ref[...]
        m = jnp.min(d, axis=1, keepdims=True)
        idx = jnp.min(jnp.where(d == m, iota, n), axis=1, keepdims=True)
        if k > 0:
            out_ref[:, k - 1:k] = idx
        d_ref[...] = jnp.where(iota == idx, jnp.inf, d)


def _knn(ref_pts):
    N = ref_pts.shape[0]
    BR = 256
    rpat = ref_pts.T
    return pl.pallas_call(
        _knn_body,
        grid=(N // BR,),
        in_specs=[
            pl.BlockSpec((BR, 3), lambda i: (i, 0)),
            pl.BlockSpec((3, N), lambda i: (0, 0)),
        ],
        out_specs=pl.BlockSpec((BR, GK), lambda i: (i, 0)),
        out_shape=jax.ShapeDtypeStruct((N, GK), jnp.int32),
        scratch_shapes=[pltpu.VMEM((BR, N), jnp.float32)],
    )(ref_pts, rpat)


def kernel(ref_node_corr_indices, src_node_corr_indices, node_corr_scores,
           ref_points_c, src_points_c, ref_feats_c, src_feats_c,
           nw1, nb1, nw2, nb2, ew1, eb1, ew2, eb2,
           uw1, ub1, uw2, ub2, ow1, ob1, ow2, ob2):
    keep = TOPK
    top_scores, top_ids = jax.lax.top_k(node_corr_scores, keep)
    ref_idx = ref_node_corr_indices[top_ids]
    src_idx = src_node_corr_indices[top_ids]
    ref_pts = ref_points_c[ref_idx]
    src_pts = src_points_c[src_idx]
    ref_f = ref_feats_c[ref_idx]
    src_f = src_feats_c[src_idx]

    num = jnp.sum(ref_f * src_f, axis=-1)
    den = jnp.maximum(jnp.linalg.norm(ref_f, axis=-1), 1e-08) * jnp.maximum(jnp.linalg.norm(src_f, axis=-1), 1e-08)
    feat_cos = (num / den)[:, None]
    feat_l2 = jnp.linalg.norm(ref_f - src_f, axis=-1, keepdims=True)
    score = jnp.clip(top_scores, MINS, None)[:, None]
    log_score = jnp.log(jnp.clip(score, MINS, None))
    node_x = jnp.concatenate([score, log_score, feat_cos, feat_l2], axis=1)
    h = _node_mlp(node_x, nw1, nb1, nw2, nb2)

    knn_ids = _knn(ref_pts)

    pts_cat = jnp.concatenate([ref_pts, src_pts, jnp.zeros((TOPK, 10), jnp.float32)], axis=1)
    nbr = pts_cat[knn_ids]
    rel = jnp.linalg.norm(ref_pts[:, None, :] - nbr[..., 0:3], axis=-1)
    sel = jnp.linalg.norm(src_pts[:, None, :] - nbr[..., 3:6], axis=-1)
    residual = jnp.abs(rel - sel)
    compat = jnp.exp(-residual ** 2 / (2.0 * SIG ** 2 + 1e-08))
    flat_ids = knn_ids.T.reshape(-1)
    h_nbr_t = _sc_gather_rows(h, flat_ids).reshape(GK, TOPK, HID)
    agg = _edge_agg(h_nbr_t, compat.T[:, :, None], residual.T[:, :, None],
                    ew1, eb1, ew2, eb2)
    h, gate2 = _update_gate(h, agg, uw1, ub1, uw2, ub2, ow1, ob1, ow2, ob2)
    gate = gate2[:, 0]

    mean_compat = compat.mean(axis=1)

    refined = jnp.clip(top_scores, MINS, None) * (0.5 * gate + 0.5 * mean_compat)
    refined = jnp.clip(refined, MINS, None)
    order = jnp.argsort(-refined)
    return (ref_idx[order], src_idx[order], refined[order])
